# Initial kernel scaffold; baseline (speedup 1.0000x reference)
import jax
import jax.numpy as jnp
from jax.experimental import pallas as pl


def kernel(x, edge_index, Wego1, bego1, Wego2, bego2, W1, a_src1, a_dst1, b1, W2, a_src2, a_dst2, b2):
    raise NotImplementedError("write your pallas kernel here")


# speedup vs baseline: 1.6579x; 1.6579x over previous
import functools

import jax
import jax.numpy as jnp
from jax.experimental import pallas as pl
from jax.experimental.pallas import tpu as pltpu

N = 10000
E = 160000
IN_DIM = 256
HID = 64
OUT_DIM = 256

_BLK = 1000


def _ego_body(x_ref, w1_ref, b1_ref, w2_ref, b2_ref, o_ref):
    h = jnp.dot(x_ref[...], w1_ref[...], preferred_element_type=jnp.float32)
    h = h + b1_ref[...]
    o = jnp.dot(h, w2_ref[...], preferred_element_type=jnp.float32)
    o_ref[...] = o + b2_ref[...]


def _ego_mlp(x, W1, b1, W2, b2):
    m, k = x.shape
    h = W1.shape[1]
    n = W2.shape[1]
    return pl.pallas_call(
        _ego_body,
        grid=(m // _BLK,),
        in_specs=[
            pl.BlockSpec((_BLK, k), lambda i: (i, 0)),
            pl.BlockSpec((k, h), lambda i: (0, 0)),
            pl.BlockSpec((h,), lambda i: (0,)),
            pl.BlockSpec((h, n), lambda i: (0, 0)),
            pl.BlockSpec((n,), lambda i: (0,)),
        ],
        out_specs=pl.BlockSpec((_BLK, n), lambda i: (i, 0)),
        out_shape=jax.ShapeDtypeStruct((m, n), jnp.float32),
    )(x, W1, b1, W2, b2)


def _pre_body(x_ref, w_ref, asrc_ref, adst_ref, h_ref, al_src_ref, al_dst_ref,
              ninit_ref, dinit_ref):
    h = jnp.dot(x_ref[...], w_ref[...], preferred_element_type=jnp.float32)
    h_ref[...] = h
    al_s = h @ asrc_ref[...]
    al_d = h @ adst_ref[...]
    al_src_ref[...] = jnp.broadcast_to(al_s[:, None], al_src_ref.shape)
    al_dst_ref[...] = jnp.broadcast_to(al_d[:, None], al_dst_ref.shape)
    e = al_s + al_d
    w_self = jnp.exp(jnp.maximum(e, 0.2 * e))
    ninit_ref[...] = w_self[:, None] * h
    dinit_ref[...] = jnp.broadcast_to(w_self[:, None], dinit_ref.shape)


def _gat_pre(x, W, a_src, a_dst):
    m, k = x.shape
    c = W.shape[1]
    return pl.pallas_call(
        _pre_body,
        grid=(m // _BLK,),
        in_specs=[
            pl.BlockSpec((_BLK, k), lambda i: (i, 0)),
            pl.BlockSpec((k, c), lambda i: (0, 0)),
            pl.BlockSpec((c,), lambda i: (0,)),
            pl.BlockSpec((c,), lambda i: (0,)),
        ],
        out_specs=[
            pl.BlockSpec((_BLK, c), lambda i: (i, 0)),
            pl.BlockSpec((_BLK, 16), lambda i: (i, 0)),
            pl.BlockSpec((_BLK, 16), lambda i: (i, 0)),
            pl.BlockSpec((_BLK, c), lambda i: (i, 0)),
            pl.BlockSpec((_BLK, 16), lambda i: (i, 0)),
        ],
        out_shape=[
            jax.ShapeDtypeStruct((m, c), jnp.float32),
            jax.ShapeDtypeStruct((m, 16), jnp.float32),
            jax.ShapeDtypeStruct((m, 16), jnp.float32),
            jax.ShapeDtypeStruct((m, c), jnp.float32),
            jax.ShapeDtypeStruct((m, 16), jnp.float32),
        ],
    )(x, W, a_src, a_dst)


def _post_body(num_ref, den_ref, b_ref, o_ref):
    o_ref[...] = num_ref[...] / (den_ref[...][:, 0:1] + 1e-16) + b_ref[...]


def _gat_post(num, den16, b):
    m, c = num.shape
    return pl.pallas_call(
        _post_body,
        grid=(m // _BLK,),
        in_specs=[
            pl.BlockSpec((_BLK, c), lambda i: (i, 0)),
            pl.BlockSpec((_BLK, 16), lambda i: (i, 0)),
            pl.BlockSpec((c,), lambda i: (0,)),
        ],
        out_specs=pl.BlockSpec((_BLK, c), lambda i: (i, 0)),
        out_shape=jax.ShapeDtypeStruct((m, c), jnp.float32),
    )(num, den16, b)


def _edge_aggregate(h, al_src, al_dst, ninit, dinit16, src, dst):
    al_src = al_src[:, 0]
    al_dst = al_dst[:, 0]
    e = al_src[src] + al_dst[dst]
    w = jnp.exp(jnp.maximum(e, 0.2 * e))
    num = ninit + jax.ops.segment_sum(w[:, None] * h[src], dst, num_segments=h.shape[0])
    den = dinit16[:, 0] + jax.ops.segment_sum(w, dst, num_segments=h.shape[0])
    return num, jnp.broadcast_to(den[:, None], (h.shape[0], 16))


def kernel(x, edge_index, Wego1, bego1, Wego2, bego2, W1, a_src1, a_dst1, b1,
           W2, a_src2, a_dst2, b2):
    src = edge_index[0]
    dst = edge_index[1]

    source = _ego_mlp(x, Wego1, bego1, Wego2, bego2)

    h1, s1, d1, ni1, di1 = _gat_pre(x, W1, a_src1, a_dst1)
    num1, den1 = _edge_aggregate(h1, s1, d1, ni1, di1, src, dst)
    out1 = _gat_post(num1, den1, b1)

    h2, s2, d2, ni2, di2 = _gat_pre(out1, W2, a_src2, a_dst2)
    num2, den2 = _edge_aggregate(h2, s2, d2, ni2, di2, src, dst)
    out2 = _gat_post(num2, den2, b2)

    return (source, out2)


# speedup vs baseline: 11.4442x; 6.9027x over previous
import dataclasses
import functools

import jax
import jax.numpy as jnp
from jax import lax
from jax.experimental import pallas as pl
from jax.experimental.pallas import tpu as pltpu
from jax.experimental.pallas import tpu_sc as plsc

N = 10000
E = 160000
IN_DIM = 256
HID = 64
OUT_DIM = 256

_BLK = 1000
_K = 128
_CH = 80
_CG = 8
_NPAD = N + 16
_NTILES = 16
_RPT_A = 632
_RPT_B = N - 15 * _RPT_A


def _ego_body(x_ref, w1_ref, b1_ref, w2_ref, b2_ref, o_ref):
    h = jnp.dot(x_ref[...], w1_ref[...], preferred_element_type=jnp.float32)
    h = h + b1_ref[...]
    o = jnp.dot(h, w2_ref[...], preferred_element_type=jnp.float32)
    o_ref[...] = o + b2_ref[...]


def _ego_mlp(x, W1, b1, W2, b2):
    m, k = x.shape
    h = W1.shape[1]
    n = W2.shape[1]
    return pl.pallas_call(
        _ego_body,
        grid=(m // _BLK,),
        in_specs=[
            pl.BlockSpec((_BLK, k), lambda i: (i, 0)),
            pl.BlockSpec((k, h), lambda i: (0, 0)),
            pl.BlockSpec((h,), lambda i: (0,)),
            pl.BlockSpec((h, n), lambda i: (0, 0)),
            pl.BlockSpec((n,), lambda i: (0,)),
        ],
        out_specs=pl.BlockSpec((_BLK, n), lambda i: (i, 0)),
        out_shape=jax.ShapeDtypeStruct((m, n), jnp.float32),
    )(x, W1, b1, W2, b2)


def _pre_body(chalf, x_ref, w_ref, asrc_ref, adst_ref, hst_ref, al_src_ref,
              al_dst_ref, ni_ref, di_ref):
    h = jnp.dot(x_ref[...], w_ref[...], preferred_element_type=jnp.float32)
    hst_ref[0] = h[:, :chalf]
    hst_ref[1] = h[:, chalf:]
    al_s = h @ asrc_ref[...]
    al_d = h @ adst_ref[...]
    al_src_ref[...] = jnp.broadcast_to(al_s[:, None], al_src_ref.shape)
    al_dst_ref[...] = jnp.broadcast_to(al_d[:, None], al_dst_ref.shape)
    e = al_s + al_d
    w_self = jnp.exp(jnp.maximum(e, 0.2 * e))
    ni = w_self[:, None] * h
    ni_ref[0] = ni[:, :chalf]
    ni_ref[1] = ni[:, chalf:]
    di_ref[...] = jnp.broadcast_to(w_self[:, None], di_ref.shape)


def _gat_pre(x, W, a_src, a_dst):
    m, k = x.shape
    c = W.shape[1]
    ch = c // 2
    return pl.pallas_call(
        functools.partial(_pre_body, ch),
        grid=(m // _BLK,),
        in_specs=[
            pl.BlockSpec((_BLK, k), lambda i: (i, 0)),
            pl.BlockSpec((k, c), lambda i: (0, 0)),
            pl.BlockSpec((c,), lambda i: (0,)),
            pl.BlockSpec((c,), lambda i: (0,)),
        ],
        out_specs=[
            pl.BlockSpec((2, _BLK, ch), lambda i: (0, i, 0)),
            pl.BlockSpec((_BLK, 16), lambda i: (i, 0)),
            pl.BlockSpec((_BLK, 16), lambda i: (i, 0)),
            pl.BlockSpec((2, _BLK, ch), lambda i: (0, i, 0)),
            pl.BlockSpec((_BLK, 16), lambda i: (i, 0)),
        ],
        out_shape=[
            jax.ShapeDtypeStruct((2, m, ch), jnp.float32),
            jax.ShapeDtypeStruct((m, 16), jnp.float32),
            jax.ShapeDtypeStruct((m, 16), jnp.float32),
            jax.ShapeDtypeStruct((2, m, ch), jnp.float32),
            jax.ShapeDtypeStruct((m, 16), jnp.float32),
        ],
    )(x, W, a_src, a_dst)


def _post_body(num_ref, den_ref, b_ref, o_ref):
    num = jnp.concatenate([num_ref[0], num_ref[1]], axis=1)
    o_ref[...] = num / (den_ref[...][:, 0:1] + 1e-16) + b_ref[...]


def _gat_post(num_st, den16, b):
    _, m, ch = num_st.shape
    c = 2 * ch
    return pl.pallas_call(
        _post_body,
        grid=(m // _BLK,),
        in_specs=[
            pl.BlockSpec((2, _BLK, ch), lambda i: (0, i, 0)),
            pl.BlockSpec((_BLK, 16), lambda i: (i, 0)),
            pl.BlockSpec((c,), lambda i: (0,)),
        ],
        out_specs=pl.BlockSpec((_BLK, c), lambda i: (i, 0)),
        out_shape=jax.ShapeDtypeStruct((m, c), jnp.float32),
    )(num_st, den16, b)


def _sc_edge_body(chalf, hst_hbm, asrc_hbm, adst_hbm, ni_hbm, di_hbm,
                  srcm_hbm, dstm_hbm, num_out, den_out,
                  asrc_t, adst_t, src_t, dst_t, rows, denr, w_t,
                  num_sh, den_sh):
    cid = lax.axis_index("c")
    sid = lax.axis_index("s")
    base = sid * _RPT_A
    coff = cid * N

    pltpu.sync_copy(asrc_hbm, asrc_t)
    pltpu.sync_copy(adst_hbm, adst_t)

    @pl.when(sid < _NTILES - 1)
    def _():
        pltpu.sync_copy(ni_hbm.at[cid, pl.ds(base, _RPT_A)],
                        num_sh.at[pl.ds(base, _RPT_A)])

        @pl.when(cid == 0)
        def _():
            pltpu.sync_copy(di_hbm.at[pl.ds(base, _RPT_A)],
                            den_sh.at[pl.ds(base, _RPT_A)])

    @pl.when(sid == _NTILES - 1)
    def _():
        pltpu.sync_copy(ni_hbm.at[cid, pl.ds(base, _RPT_B)],
                        num_sh.at[pl.ds(base, _RPT_B)])

        @pl.when(cid == 0)
        def _():
            pltpu.sync_copy(di_hbm.at[pl.ds(base, _RPT_B)],
                            den_sh.at[pl.ds(base, _RPT_B)])

    plsc.subcore_barrier()

    @pl.loop(0, _CH // _CG)
    def _group(g0):
        pltpu.sync_copy(srcm_hbm.at[sid, pl.ds(g0 * _CG, _CG)], src_t)
        pltpu.sync_copy(dstm_hbm.at[sid, pl.ds(g0 * _CG, _CG)], dst_t)

        @pl.loop(0, _CG)
        def _chunk(c):
            srow = src_t.at[c]
            drow = dst_t.at[c]

            @pl.loop(0, _K, step=16)
            def _wgrp(g):
                s16 = srow[pl.ds(g, 16)]
                d16 = drow[pl.ds(g, 16)]
                av = plsc.load_gather(asrc_t, [s16])
                bv = plsc.load_gather(adst_t, [d16])
                e = av + bv
                e = jnp.maximum(e, 0.2 * e)
                w_t[pl.ds(g, 16)] = jnp.exp(e)
                srow[pl.ds(g, 16)] = s16 + coff

            pltpu.sync_copy(hst_hbm.at[srow], rows)

            @pl.loop(0, _K)
            def _row(i):
                iv = lax.broadcast(i, (16,))
                wv = plsc.load_gather(w_t, [iv])
                denr[i, :] = wv
                for cc in range(chalf // 16):
                    sl = (i, pl.ds(cc * 16, 16))
                    rows[sl] = rows[sl] * wv

            pltpu.sync_copy(rows, num_sh.at[drow], add=True)

            @pl.when(cid == 0)
            def _():
                pltpu.sync_copy(denr, den_sh.at[drow], add=True)

    plsc.subcore_barrier()

    @pl.when(sid < _NTILES - 1)
    def _():
        pltpu.sync_copy(num_sh.at[pl.ds(base, _RPT_A)],
                        num_out.at[cid, pl.ds(base, _RPT_A)])

        @pl.when(cid == 0)
        def _():
            pltpu.sync_copy(den_sh.at[pl.ds(base, _RPT_A)],
                            den_out.at[pl.ds(base, _RPT_A)])

    @pl.when(sid == _NTILES - 1)
    def _():
        pltpu.sync_copy(num_sh.at[pl.ds(base, _RPT_B)],
                        num_out.at[cid, pl.ds(base, _RPT_B)])

        @pl.when(cid == 0)
        def _():
            pltpu.sync_copy(den_sh.at[pl.ds(base, _RPT_B)],
                            den_out.at[pl.ds(base, _RPT_B)])


def _sc_edge(hst, asrc, adst, ni_st, di, srcm, dstm, chalf):
    mesh = plsc.VectorSubcoreMesh(core_axis_name="c", subcore_axis_name="s")
    cp = pltpu.CompilerParams()
    if "needs_layout_passes" in pltpu.CompilerParams.__dataclass_fields__:
        cp = dataclasses.replace(cp, needs_layout_passes=False)
    if "use_tc_tiling_on_sc" in pltpu.CompilerParams.__dataclass_fields__:
        cp = dataclasses.replace(cp, use_tc_tiling_on_sc=False)
    kern = pl.kernel(
        functools.partial(_sc_edge_body, chalf),
        mesh=mesh,
        compiler_params=cp,
        out_type=[
            jax.ShapeDtypeStruct((2, N, chalf), jnp.float32),
            jax.ShapeDtypeStruct((N, 16), jnp.float32),
        ],
        scratch_types=[
            pltpu.VMEM((_NPAD,), jnp.float32),
            pltpu.VMEM((_NPAD,), jnp.float32),
            pltpu.VMEM((_CG, _K), jnp.int32),
            pltpu.VMEM((_CG, _K), jnp.int32),
            pltpu.VMEM((_K, chalf), jnp.float32),
            pltpu.VMEM((_K, 16), jnp.float32),
            pltpu.VMEM((_K,), jnp.float32),
            pltpu.VMEM_SHARED((_NPAD, chalf), jnp.float32),
            pltpu.VMEM_SHARED((_NPAD, 16), jnp.float32),
        ],
    )
    return kern(hst, asrc, adst, ni_st, di, srcm, dstm)


def _gat_layer(x, edge_tables, W, a_src, a_dst, b):
    srcm, dstm = edge_tables
    chalf = W.shape[1] // 2
    hst, al_s, al_d, ni_st, di = _gat_pre(x, W, a_src, a_dst)
    asrc_flat = jnp.pad(al_s[:, 0], (0, _NPAD - N))
    adst_flat = jnp.pad(al_d[:, 0], (0, _NPAD - N))
    num_st, den16 = _sc_edge(hst.reshape(2 * N, chalf), asrc_flat, adst_flat,
                             ni_st, di, srcm, dstm, chalf)
    return _gat_post(num_st, den16, b)


def kernel(x, edge_index, Wego1, bego1, Wego2, bego2, W1, a_src1, a_dst1, b1,
           W2, a_src2, a_dst2, b2):
    pad = _NTILES * _CH * _K - E
    srcm = jnp.concatenate(
        [edge_index[0], jnp.zeros((pad,), jnp.int32)]).reshape(_NTILES, _CH, _K)
    dstm = jnp.concatenate(
        [edge_index[1], jnp.full((pad,), N, jnp.int32)]).reshape(_NTILES, _CH, _K)

    source = _ego_mlp(x, Wego1, bego1, Wego2, bego2)
    out1 = _gat_layer(x, (srcm, dstm), W1, a_src1, a_dst1, b1)
    out2 = _gat_layer(out1, (srcm, dstm), W2, a_src2, a_dst2, b2)
    return (source, out2)


# speedup vs baseline: 14.0923x; 1.2314x over previous
import dataclasses
import functools

import jax
import jax.numpy as jnp
from jax import lax
from jax.experimental import pallas as pl
from jax.experimental.pallas import tpu as pltpu
from jax.experimental.pallas import tpu_sc as plsc

N = 10000
E = 160000
IN_DIM = 256
HID = 64
OUT_DIM = 256

_BLK = 1000
_K = 128
_CH = 80
_CG = 8
_NPAD = N + 16
_NTILES = 16
_RPT_A = 632
_RPT_B = N - 15 * _RPT_A


def _ego_body(x_ref, w1_ref, b1_ref, w2_ref, b2_ref, o_ref):
    h = jnp.dot(x_ref[...], w1_ref[...], preferred_element_type=jnp.float32)
    h = h + b1_ref[...]
    o = jnp.dot(h, w2_ref[...], preferred_element_type=jnp.float32)
    o_ref[...] = o + b2_ref[...]


def _ego_mlp(x, W1, b1, W2, b2):
    m, k = x.shape
    h = W1.shape[1]
    n = W2.shape[1]
    return pl.pallas_call(
        _ego_body,
        grid=(m // _BLK,),
        in_specs=[
            pl.BlockSpec((_BLK, k), lambda i: (i, 0)),
            pl.BlockSpec((k, h), lambda i: (0, 0)),
            pl.BlockSpec((h,), lambda i: (0,)),
            pl.BlockSpec((h, n), lambda i: (0, 0)),
            pl.BlockSpec((n,), lambda i: (0,)),
        ],
        out_specs=pl.BlockSpec((_BLK, n), lambda i: (i, 0)),
        out_shape=jax.ShapeDtypeStruct((m, n), jnp.float32),
    )(x, W1, b1, W2, b2)


def _pre_body(chalf, x_ref, w_ref, asrc_ref, adst_ref, hst_ref, al_src_ref,
              al_dst_ref, ni_ref, di_ref):
    h = jnp.dot(x_ref[...], w_ref[...], preferred_element_type=jnp.float32)
    hst_ref[0] = h[:, :chalf]
    hst_ref[1] = h[:, chalf:]
    al_s = h @ asrc_ref[...]
    al_d = h @ adst_ref[...]
    al_src_ref[...] = jnp.broadcast_to(al_s[:, None], al_src_ref.shape)
    al_dst_ref[...] = jnp.broadcast_to(al_d[:, None], al_dst_ref.shape)
    e = al_s + al_d
    w_self = jnp.exp(jnp.maximum(e, 0.2 * e))
    ni = w_self[:, None] * h
    ni_ref[0] = ni[:, :chalf]
    ni_ref[1] = ni[:, chalf:]
    di_ref[...] = jnp.broadcast_to(w_self[:, None], di_ref.shape)


def _gat_pre(x, W, a_src, a_dst):
    m, k = x.shape
    c = W.shape[1]
    ch = c // 2
    return pl.pallas_call(
        functools.partial(_pre_body, ch),
        grid=(m // _BLK,),
        in_specs=[
            pl.BlockSpec((_BLK, k), lambda i: (i, 0)),
            pl.BlockSpec((k, c), lambda i: (0, 0)),
            pl.BlockSpec((c,), lambda i: (0,)),
            pl.BlockSpec((c,), lambda i: (0,)),
        ],
        out_specs=[
            pl.BlockSpec((2, _BLK, ch), lambda i: (0, i, 0)),
            pl.BlockSpec((_BLK, 16), lambda i: (i, 0)),
            pl.BlockSpec((_BLK, 16), lambda i: (i, 0)),
            pl.BlockSpec((2, _BLK, ch), lambda i: (0, i, 0)),
            pl.BlockSpec((_BLK, 16), lambda i: (i, 0)),
        ],
        out_shape=[
            jax.ShapeDtypeStruct((2, m, ch), jnp.float32),
            jax.ShapeDtypeStruct((m, 16), jnp.float32),
            jax.ShapeDtypeStruct((m, 16), jnp.float32),
            jax.ShapeDtypeStruct((2, m, ch), jnp.float32),
            jax.ShapeDtypeStruct((m, 16), jnp.float32),
        ],
    )(x, W, a_src, a_dst)


def _post_body(num_ref, den_ref, b_ref, o_ref):
    num = jnp.concatenate([num_ref[0], num_ref[1]], axis=1)
    o_ref[...] = num / (den_ref[...][:, 0:1] + 1e-16) + b_ref[...]


def _gat_post(num_st, den16, b):
    _, m, ch = num_st.shape
    c = 2 * ch
    return pl.pallas_call(
        _post_body,
        grid=(m // _BLK,),
        in_specs=[
            pl.BlockSpec((2, _BLK, ch), lambda i: (0, i, 0)),
            pl.BlockSpec((_BLK, 16), lambda i: (i, 0)),
            pl.BlockSpec((c,), lambda i: (0,)),
        ],
        out_specs=pl.BlockSpec((_BLK, c), lambda i: (i, 0)),
        out_shape=jax.ShapeDtypeStruct((m, c), jnp.float32),
    )(num_st, den16, b)


def _sc_edge_body(chalf, hst_hbm, asrc_hbm, adst_hbm, ni_hbm, di_hbm,
                  srcm_hbm, dstm_hbm, num_out, den_out,
                  asrc_t, adst_t, src_t, dst_t, rows, denr, w_t,
                  num_sh, den_sh, nsem, dsem):
    cid = lax.axis_index("c")
    sid = lax.axis_index("s")
    base = sid * _RPT_A
    coff = cid * N

    pltpu.sync_copy(asrc_hbm, asrc_t)
    pltpu.sync_copy(adst_hbm, adst_t)

    @pl.when(sid < _NTILES - 1)
    def _():
        pltpu.sync_copy(ni_hbm.at[cid, pl.ds(base, _RPT_A)],
                        num_sh.at[pl.ds(base, _RPT_A)])

        @pl.when(cid == 0)
        def _():
            pltpu.sync_copy(di_hbm.at[pl.ds(base, _RPT_A)],
                            den_sh.at[pl.ds(base, _RPT_A)])

    @pl.when(sid == _NTILES - 1)
    def _():
        pltpu.sync_copy(ni_hbm.at[cid, pl.ds(base, _RPT_B)],
                        num_sh.at[pl.ds(base, _RPT_B)])

        @pl.when(cid == 0)
        def _():
            pltpu.sync_copy(di_hbm.at[pl.ds(base, _RPT_B)],
                            den_sh.at[pl.ds(base, _RPT_B)])

    plsc.subcore_barrier()

    def _drain_scatters():
        pltpu.make_async_copy(hst_hbm.at[pl.ds(0, _K)], rows, nsem).wait()

        @pl.when(cid == 0)
        def _():
            pltpu.make_async_copy(di_hbm.at[pl.ds(0, _K)], denr, dsem).wait()

    @pl.loop(0, _CH // _CG)
    def _group(g0):
        @pl.when(g0 > 0)
        def _():
            _drain_scatters()

        pltpu.sync_copy(srcm_hbm.at[sid, pl.ds(g0 * _CG, _CG)], src_t)
        pltpu.sync_copy(dstm_hbm.at[sid, pl.ds(g0 * _CG, _CG)], dst_t)

        @pl.loop(0, _CG)
        def _chunk(c):
            srow = src_t.at[c]
            drow = dst_t.at[c]

            @pl.loop(0, _K, step=16, unroll=True)
            def _wgrp(g):
                s16 = srow[pl.ds(g, 16)]
                d16 = drow[pl.ds(g, 16)]
                av = plsc.load_gather(asrc_t, [s16])
                bv = plsc.load_gather(adst_t, [d16])
                e = av + bv
                e = jnp.maximum(e, 0.2 * e)
                w_t[pl.ds(g, 16)] = jnp.exp(e)
                srow[pl.ds(g, 16)] = s16 + coff

            @pl.when(c > 0)
            def _():
                _drain_scatters()

            pltpu.sync_copy(hst_hbm.at[srow], rows)

            @plsc.parallel_loop(0, _K, unroll=4)
            def _row(i):
                iv = lax.broadcast(i, (16,))
                wv = plsc.load_gather(w_t, [iv])
                denr[i, :] = wv
                for cc in range(chalf // 16):
                    sl = (i, pl.ds(cc * 16, 16))
                    rows[sl] = rows[sl] * wv

            pltpu.async_copy(rows, num_sh.at[drow], nsem, add=True)

            @pl.when(cid == 0)
            def _():
                pltpu.async_copy(denr, den_sh.at[drow], dsem, add=True)

    _drain_scatters()
    plsc.subcore_barrier()

    @pl.when(sid < _NTILES - 1)
    def _():
        pltpu.sync_copy(num_sh.at[pl.ds(base, _RPT_A)],
                        num_out.at[cid, pl.ds(base, _RPT_A)])

        @pl.when(cid == 0)
        def _():
            pltpu.sync_copy(den_sh.at[pl.ds(base, _RPT_A)],
                            den_out.at[pl.ds(base, _RPT_A)])

    @pl.when(sid == _NTILES - 1)
    def _():
        pltpu.sync_copy(num_sh.at[pl.ds(base, _RPT_B)],
                        num_out.at[cid, pl.ds(base, _RPT_B)])

        @pl.when(cid == 0)
        def _():
            pltpu.sync_copy(den_sh.at[pl.ds(base, _RPT_B)],
                            den_out.at[pl.ds(base, _RPT_B)])


def _sc_edge(hst, asrc, adst, ni_st, di, srcm, dstm, chalf):
    mesh = plsc.VectorSubcoreMesh(core_axis_name="c", subcore_axis_name="s")
    cp = pltpu.CompilerParams()
    if "needs_layout_passes" in pltpu.CompilerParams.__dataclass_fields__:
        cp = dataclasses.replace(cp, needs_layout_passes=False)
    if "use_tc_tiling_on_sc" in pltpu.CompilerParams.__dataclass_fields__:
        cp = dataclasses.replace(cp, use_tc_tiling_on_sc=False)
    kern = pl.kernel(
        functools.partial(_sc_edge_body, chalf),
        mesh=mesh,
        compiler_params=cp,
        out_type=[
            jax.ShapeDtypeStruct((2, N, chalf), jnp.float32),
            jax.ShapeDtypeStruct((N, 16), jnp.float32),
        ],
        scratch_types=[
            pltpu.VMEM((_NPAD,), jnp.float32),
            pltpu.VMEM((_NPAD,), jnp.float32),
            pltpu.VMEM((_CG, _K), jnp.int32),
            pltpu.VMEM((_CG, _K), jnp.int32),
            pltpu.VMEM((_K, chalf), jnp.float32),
            pltpu.VMEM((_K, 16), jnp.float32),
            pltpu.VMEM((_K,), jnp.float32),
            pltpu.VMEM_SHARED((_NPAD, chalf), jnp.float32),
            pltpu.VMEM_SHARED((_NPAD, 16), jnp.float32),
            pltpu.SemaphoreType.DMA,
            pltpu.SemaphoreType.DMA,
        ],
    )
    return kern(hst, asrc, adst, ni_st, di, srcm, dstm)


def _gat_layer(x, edge_tables, W, a_src, a_dst, b):
    srcm, dstm = edge_tables
    chalf = W.shape[1] // 2
    hst, al_s, al_d, ni_st, di = _gat_pre(x, W, a_src, a_dst)
    asrc_flat = jnp.pad(al_s[:, 0], (0, _NPAD - N))
    adst_flat = jnp.pad(al_d[:, 0], (0, _NPAD - N))
    num_st, den16 = _sc_edge(hst.reshape(2 * N, chalf), asrc_flat, adst_flat,
                             ni_st, di, srcm, dstm, chalf)
    return _gat_post(num_st, den16, b)


def kernel(x, edge_index, Wego1, bego1, Wego2, bego2, W1, a_src1, a_dst1, b1,
           W2, a_src2, a_dst2, b2):
    pad = _NTILES * _CH * _K - E
    srcm = jnp.concatenate(
        [edge_index[0], jnp.zeros((pad,), jnp.int32)]).reshape(_NTILES, _CH, _K)
    dstm = jnp.concatenate(
        [edge_index[1], jnp.full((pad,), N, jnp.int32)]).reshape(_NTILES, _CH, _K)

    source = _ego_mlp(x, Wego1, bego1, Wego2, bego2)
    out1 = _gat_layer(x, (srcm, dstm), W1, a_src1, a_dst1, b1)
    out2 = _gat_layer(out1, (srcm, dstm), W2, a_src2, a_dst2, b2)
    return (source, out2)


# speedup vs baseline: 16.2585x; 1.1537x over previous
import dataclasses
import functools

import jax
import jax.numpy as jnp
from jax import lax
from jax.experimental import pallas as pl
from jax.experimental.pallas import tpu as pltpu
from jax.experimental.pallas import tpu_sc as plsc

N = 10000
E = 160000
IN_DIM = 256
HID = 64
OUT_DIM = 256

_BLK = 1000
_K = 64
_NCH = 160
_CG = 8
_NPAD = N + 16
_NTILES = 16
_RPT_A = 632
_RPT_B = N - 15 * _RPT_A


def _ego_body(x_ref, w1_ref, b1_ref, w2_ref, b2_ref, o_ref):
    h = jnp.dot(x_ref[...], w1_ref[...], preferred_element_type=jnp.float32)
    h = h + b1_ref[...]
    o = jnp.dot(h, w2_ref[...], preferred_element_type=jnp.float32)
    o_ref[...] = o + b2_ref[...]


def _ego_mlp(x, W1, b1, W2, b2):
    m, k = x.shape
    h = W1.shape[1]
    n = W2.shape[1]
    return pl.pallas_call(
        _ego_body,
        grid=(m // _BLK,),
        in_specs=[
            pl.BlockSpec((_BLK, k), lambda i: (i, 0)),
            pl.BlockSpec((k, h), lambda i: (0, 0)),
            pl.BlockSpec((h,), lambda i: (0,)),
            pl.BlockSpec((h, n), lambda i: (0, 0)),
            pl.BlockSpec((n,), lambda i: (0,)),
        ],
        out_specs=pl.BlockSpec((_BLK, n), lambda i: (i, 0)),
        out_shape=jax.ShapeDtypeStruct((m, n), jnp.float32),
    )(x, W1, b1, W2, b2)


def _pre_body(chalf, x_ref, w_ref, asrc_ref, adst_ref, hst_ref, al_src_ref,
              al_dst_ref, ni_ref, di_ref):
    h = jnp.dot(x_ref[...], w_ref[...], preferred_element_type=jnp.float32)
    hst_ref[0] = h[:, :chalf]
    hst_ref[1] = h[:, chalf:]
    al_s = h @ asrc_ref[...]
    al_d = h @ adst_ref[...]
    al_src_ref[...] = jnp.broadcast_to(al_s[:, None], al_src_ref.shape)
    al_dst_ref[...] = jnp.broadcast_to(al_d[:, None], al_dst_ref.shape)
    e = al_s + al_d
    w_self = jnp.exp(jnp.maximum(e, 0.2 * e))
    ni = w_self[:, None] * h
    ni_ref[0] = ni[:, :chalf]
    ni_ref[1] = ni[:, chalf:]
    di_ref[...] = jnp.broadcast_to(w_self[:, None], di_ref.shape)


def _gat_pre(x, W, a_src, a_dst):
    m, k = x.shape
    c = W.shape[1]
    ch = c // 2
    return pl.pallas_call(
        functools.partial(_pre_body, ch),
        grid=(m // _BLK,),
        in_specs=[
            pl.BlockSpec((_BLK, k), lambda i: (i, 0)),
            pl.BlockSpec((k, c), lambda i: (0, 0)),
            pl.BlockSpec((c,), lambda i: (0,)),
            pl.BlockSpec((c,), lambda i: (0,)),
        ],
        out_specs=[
            pl.BlockSpec((2, _BLK, ch), lambda i: (0, i, 0)),
            pl.BlockSpec((_BLK, 16), lambda i: (i, 0)),
            pl.BlockSpec((_BLK, 16), lambda i: (i, 0)),
            pl.BlockSpec((2, _BLK, ch), lambda i: (0, i, 0)),
            pl.BlockSpec((_BLK, 16), lambda i: (i, 0)),
        ],
        out_shape=[
            jax.ShapeDtypeStruct((2, m, ch), jnp.float32),
            jax.ShapeDtypeStruct((m, 16), jnp.float32),
            jax.ShapeDtypeStruct((m, 16), jnp.float32),
            jax.ShapeDtypeStruct((2, m, ch), jnp.float32),
            jax.ShapeDtypeStruct((m, 16), jnp.float32),
        ],
    )(x, W, a_src, a_dst)


def _post_body(num_ref, den_ref, di_ref, b_ref, o_ref):
    num = jnp.concatenate([num_ref[0], num_ref[1]], axis=1)
    den = den_ref[0] + den_ref[1] - di_ref[...]
    o_ref[...] = num / (den[:, 0:1] + 1e-16) + b_ref[...]


def _gat_post(num_st, den2, di, b):
    _, m, ch = num_st.shape
    c = 2 * ch
    return pl.pallas_call(
        _post_body,
        grid=(m // _BLK,),
        in_specs=[
            pl.BlockSpec((2, _BLK, ch), lambda i: (0, i, 0)),
            pl.BlockSpec((2, _BLK, 16), lambda i: (0, i, 0)),
            pl.BlockSpec((_BLK, 16), lambda i: (i, 0)),
            pl.BlockSpec((c,), lambda i: (0,)),
        ],
        out_specs=pl.BlockSpec((_BLK, c), lambda i: (i, 0)),
        out_shape=jax.ShapeDtypeStruct((m, c), jnp.float32),
    )(num_st, den2, di, b)


def _sc_edge_body(chalf, hst_hbm, asrc_hbm, adst_hbm, ni_hbm, di_hbm,
                  srcm_hbm, dstm_hbm, num_out, den_out,
                  asrc_t, adst_t, src_t, dst_t, dsave,
                  rows0, rows1, denr0, denr1, w0, w1,
                  num_sh, den_sh, gsem0, gsem1, ssem0, ssem1):
    cid = lax.axis_index("c")
    sid = lax.axis_index("s")
    base = sid * _RPT_A
    coff = cid * N

    pltpu.sync_copy(asrc_hbm, asrc_t)
    pltpu.sync_copy(adst_hbm, adst_t)
    pltpu.sync_copy(srcm_hbm.at[sid, pl.ds(0, _CG)], src_t)
    pltpu.sync_copy(dstm_hbm.at[sid, pl.ds(0, _CG)], dst_t)

    @pl.when(sid < _NTILES - 1)
    def _():
        pltpu.sync_copy(ni_hbm.at[cid, pl.ds(base, _RPT_A)],
                        num_sh.at[pl.ds(base, _RPT_A)])
        pltpu.sync_copy(di_hbm.at[pl.ds(base, _RPT_A)],
                        den_sh.at[pl.ds(base, _RPT_A)])

    @pl.when(sid == _NTILES - 1)
    def _():
        pltpu.sync_copy(ni_hbm.at[cid, pl.ds(base, _RPT_B)],
                        num_sh.at[pl.ds(base, _RPT_B)])
        pltpu.sync_copy(di_hbm.at[pl.ds(base, _RPT_B)],
                        den_sh.at[pl.ds(base, _RPT_B)])

    plsc.subcore_barrier()

    def _wgrp(j, wbuf, dpar):
        srow = src_t.at[j]
        drow = dst_t.at[j]

        @pl.loop(0, _K, step=16, unroll=True)
        def _(g):
            s16 = srow[pl.ds(g, 16)]
            d16 = drow[pl.ds(g, 16)]
            av = plsc.load_gather(asrc_t, [s16])
            bv = plsc.load_gather(adst_t, [d16])
            e = av + bv
            e = jnp.maximum(e, 0.2 * e)
            wbuf[pl.ds(g, 16)] = jnp.exp(e)
            srow[pl.ds(g, 16)] = s16 + coff
            dsave[dpar, pl.ds(g, 16)] = d16

    def _scale(rowsb, wbuf, denb):
        @plsc.parallel_loop(0, _K, unroll=4)
        def _(i):
            iv = lax.broadcast(i, (16,))
            wv = plsc.load_gather(wbuf, [iv])
            denb[i, :] = wv
            for cc in range(chalf // 16):
                sl = (i, pl.ds(cc * 16, 16))
                rowsb[sl] = rowsb[sl] * wv

    def _half(c, parx, rowsx, denrx, wx, gsemx, ssemx,
              rowsy, denry, wy, gsemy, ssemy):
        pary = 1 - parx

        @pl.when(c > 0)
        def _():
            pltpu.make_async_copy(hst_hbm.at[pl.ds(0, _K)], rowsy, ssemy).wait()

            @pl.when(cid == pary)
            def _():
                pltpu.make_async_copy(di_hbm.at[pl.ds(0, _K)], denry,
                                      ssemy).wait()

        nxt = c + 1
        bnd = jnp.logical_and(nxt % _CG == 0, nxt < _NCH)

        @pl.when(bnd)
        def _():
            pltpu.make_async_copy(hst_hbm.at[pl.ds(0, _K)], rowsx, gsemx).wait()
            pltpu.sync_copy(srcm_hbm.at[sid, pl.ds(nxt, _CG)], src_t)
            pltpu.sync_copy(dstm_hbm.at[sid, pl.ds(nxt, _CG)], dst_t)

        @pl.when(nxt < _NCH)
        def _():
            jn = lax.rem(nxt, _CG)
            _wgrp(jn, wy, pary)
            pltpu.async_copy(hst_hbm.at[src_t.at[jn]], rowsy, gsemy)

        @pl.when(jnp.logical_not(bnd))
        def _():
            pltpu.make_async_copy(hst_hbm.at[pl.ds(0, _K)], rowsx, gsemx).wait()

        _scale(rowsx, wx, denrx)

        pltpu.async_copy(rowsx, num_sh.at[dsave.at[parx]], ssemx, add=True)

        @pl.when(cid == parx)
        def _():
            pltpu.async_copy(denrx, den_sh.at[dsave.at[parx]], ssemx, add=True)

    _wgrp(0, w0, 0)
    pltpu.async_copy(hst_hbm.at[src_t.at[0]], rows0, gsem0)

    @pl.loop(0, _NCH, step=2)
    def _pair(c):
        _half(c, 0, rows0, denr0, w0, gsem0, ssem0,
              rows1, denr1, w1, gsem1, ssem1)
        _half(c + 1, 1, rows1, denr1, w1, gsem1, ssem1,
              rows0, denr0, w0, gsem0, ssem0)

    pltpu.make_async_copy(hst_hbm.at[pl.ds(0, _K)], rows1, ssem1).wait()

    @pl.when(cid == 1)
    def _():
        pltpu.make_async_copy(di_hbm.at[pl.ds(0, _K)], denr1, ssem1).wait()

    plsc.subcore_barrier()

    @pl.when(sid < _NTILES - 1)
    def _():
        pltpu.sync_copy(num_sh.at[pl.ds(base, _RPT_A)],
                        num_out.at[cid, pl.ds(base, _RPT_A)])
        pltpu.sync_copy(den_sh.at[pl.ds(base, _RPT_A)],
                        den_out.at[cid, pl.ds(base, _RPT_A)])

    @pl.when(sid == _NTILES - 1)
    def _():
        pltpu.sync_copy(num_sh.at[pl.ds(base, _RPT_B)],
                        num_out.at[cid, pl.ds(base, _RPT_B)])
        pltpu.sync_copy(den_sh.at[pl.ds(base, _RPT_B)],
                        den_out.at[cid, pl.ds(base, _RPT_B)])


def _sc_edge(hst, asrc, adst, ni_st, di, srcm, dstm, chalf):
    mesh = plsc.VectorSubcoreMesh(core_axis_name="c", subcore_axis_name="s")
    cp = pltpu.CompilerParams()
    if "needs_layout_passes" in pltpu.CompilerParams.__dataclass_fields__:
        cp = dataclasses.replace(cp, needs_layout_passes=False)
    if "use_tc_tiling_on_sc" in pltpu.CompilerParams.__dataclass_fields__:
        cp = dataclasses.replace(cp, use_tc_tiling_on_sc=False)
    kern = pl.kernel(
        functools.partial(_sc_edge_body, chalf),
        mesh=mesh,
        compiler_params=cp,
        out_type=[
            jax.ShapeDtypeStruct((2, N, chalf), jnp.float32),
            jax.ShapeDtypeStruct((2, N, 16), jnp.float32),
        ],
        scratch_types=[
            pltpu.VMEM((_NPAD,), jnp.float32),
            pltpu.VMEM((_NPAD,), jnp.float32),
            pltpu.VMEM((_CG, _K), jnp.int32),
            pltpu.VMEM((_CG, _K), jnp.int32),
            pltpu.VMEM((2, _K), jnp.int32),
            pltpu.VMEM((_K, chalf), jnp.float32),
            pltpu.VMEM((_K, chalf), jnp.float32),
            pltpu.VMEM((_K, 16), jnp.float32),
            pltpu.VMEM((_K, 16), jnp.float32),
            pltpu.VMEM((_K,), jnp.float32),
            pltpu.VMEM((_K,), jnp.float32),
            pltpu.VMEM_SHARED((_NPAD, chalf), jnp.float32),
            pltpu.VMEM_SHARED((_NPAD, 16), jnp.float32),
            pltpu.SemaphoreType.DMA,
            pltpu.SemaphoreType.DMA,
            pltpu.SemaphoreType.DMA,
            pltpu.SemaphoreType.DMA,
        ],
    )
    return kern(hst, asrc, adst, ni_st, di, srcm, dstm)


def _gat_layer(x, edge_tables, W, a_src, a_dst, b):
    srcm, dstm = edge_tables
    chalf = W.shape[1] // 2
    hst, al_s, al_d, ni_st, di = _gat_pre(x, W, a_src, a_dst)
    asrc_flat = jnp.pad(al_s[:, 0], (0, _NPAD - N))
    adst_flat = jnp.pad(al_d[:, 0], (0, _NPAD - N))
    num_st, den2 = _sc_edge(hst.reshape(2 * N, chalf), asrc_flat, adst_flat,
                            ni_st, di, srcm, dstm, chalf)
    return _gat_post(num_st, den2, di, b)


def kernel(x, edge_index, Wego1, bego1, Wego2, bego2, W1, a_src1, a_dst1, b1,
           W2, a_src2, a_dst2, b2):
    pad = _NTILES * _NCH * _K - E
    srcm = jnp.concatenate(
        [edge_index[0], jnp.zeros((pad,), jnp.int32)]).reshape(_NTILES, _NCH, _K)
    dstm = jnp.concatenate(
        [edge_index[1], jnp.full((pad,), N, jnp.int32)]).reshape(_NTILES, _NCH, _K)

    source = _ego_mlp(x, Wego1, bego1, Wego2, bego2)
    out1 = _gat_layer(x, (srcm, dstm), W1, a_src1, a_dst1, b1)
    out2 = _gat_layer(out1, (srcm, dstm), W2, a_src2, a_dst2, b2)
    return (source, out2)
